# Initial kernel scaffold; baseline (speedup 1.0000x reference)
import jax
import jax.numpy as jnp
from jax.experimental import pallas as pl


def kernel(t, x_input, W1, g1, b1, W2, g2, b2):
    raise NotImplementedError("write your pallas kernel here")


# speedup vs baseline: 2.6954x; 2.6954x over previous
import functools

import jax
import jax.numpy as jnp
from jax import lax
from jax.experimental import pallas as pl
from jax.experimental.pallas import tpu as pltpu
from jax.experimental.pallas import tpu_sc as plsc

B, DIMS, N = 4, 512, 512
H = DIMS // 2
K = 15
EPS = 1e-5

NC, NS = 2, 16
NW = NC * NS
PTS_W = (B * N) // NW
SUB = 4
NSUB = PTS_W // SUB


def _ka_body(x_ref, a_ref, b_ref, gt_ref, cct_ref, xt_ref, idx_ref):
    bb = pl.program_id(0)
    X = x_ref[0]
    Xp = X[H:, :]
    inner = lax.dot_general(Xp, Xp, (((0,), (0,)), ((), ())),
                            preferred_element_type=jnp.float32)
    xx = jnp.sum(Xp * Xp, axis=0, keepdims=True)
    P = 2.0 * inner - xx - jnp.transpose(xx)

    iota_j = lax.broadcasted_iota(jnp.int32, (N, N), 1)
    neg = jnp.float32(-jnp.inf)
    cols = []
    for _ in range(K + 1):
        rm = jnp.max(P, axis=1, keepdims=True)
        cand = jnp.where(P == rm, iota_j, jnp.int32(N))
        am = jnp.min(cand, axis=1, keepdims=True)
        P = jnp.where(iota_j == am, neg, P)
        cols.append(am)
    idx_ref[...] = jnp.concatenate(cols, axis=1) + bb * N

    gt_ref[0] = lax.dot_general(X, a_ref[...], (((0,), (1,)), ((), ())),
                                preferred_element_type=jnp.float32)
    cct_ref[0] = lax.dot_general(X, b_ref[...], (((0,), (1,)), ((), ())),
                                 preferred_element_type=jnp.float32)
    xt_ref[0] = jnp.transpose(X)


def _run_a(x, A, Bm):
    return pl.pallas_call(
        _ka_body,
        grid=(B,),
        in_specs=[
            pl.BlockSpec((1, DIMS, N), lambda b: (b, 0, 0)),
            pl.BlockSpec((DIMS, DIMS), lambda b: (0, 0)),
            pl.BlockSpec((DIMS, DIMS), lambda b: (0, 0)),
        ],
        out_specs=[
            pl.BlockSpec((1, N, DIMS), lambda b: (b, 0, 0)),
            pl.BlockSpec((1, N, DIMS), lambda b: (b, 0, 0)),
            pl.BlockSpec((1, N, DIMS), lambda b: (b, 0, 0)),
            pl.BlockSpec((N, K + 1), lambda b: (b, 0)),
        ],
        out_shape=[
            jax.ShapeDtypeStruct((B, N, DIMS), jnp.float32),
            jax.ShapeDtypeStruct((B, N, DIMS), jnp.float32),
            jax.ShapeDtypeStruct((B, N, DIMS), jnp.float32),
            jax.ShapeDtypeStruct((B * N, K + 1), jnp.int32),
        ],
    )(x, A, Bm)


def _sc_body(gt_hbm, xt_hbm, idx_hbm,
             s1_hbm, s2_hbm, mg_hbm, mn_hbm, mx_hbm,
             idx_v, rg_v, rx_v, o1_v, o2_v, o3_v, o4_v, o5_v, semg, semx):
    wid = lax.axis_index("s") * NC + lax.axis_index("c")

    def sub_step(s, carry):
        p0 = wid * PTS_W + s * SUB
        pltpu.sync_copy(idx_hbm.at[pl.ds(p0 * (K + 1), SUB * (K + 1))], idx_v)
        cg = pltpu.async_copy(gt_hbm.at[idx_v], rg_v, semg)
        cx = pltpu.async_copy(xt_hbm.at[idx_v], rx_v, semx)
        cg.wait()
        cx.wait()

        def cb_step(cb, c2):
            o = pl.ds(cb * 16, 16)
            for p in range(SUB):
                r0 = p * (K + 1)
                v = rg_v[r0 + 1, o]
                s1 = v
                s2 = v * v
                vmax = v
                vmin = v
                for kk in range(2, K + 1):
                    v = rg_v[r0 + kk, o]
                    s1 = s1 + v
                    s2 = s2 + v * v
                    vmax = jnp.maximum(vmax, v)
                    vmin = jnp.minimum(vmin, v)
                u = rx_v[r0 + 1, o]
                umax = u
                for kk in range(2, K + 1):
                    umax = jnp.maximum(umax, rx_v[r0 + kk, o])
                o1_v[p, o] = s1
                o2_v[p, o] = s2
                o3_v[p, o] = vmax
                o4_v[p, o] = vmin
                o5_v[p, o] = umax
            return c2

        lax.fori_loop(0, DIMS // 16, cb_step, 0)
        pltpu.sync_copy(o1_v, s1_hbm.at[pl.ds(p0, SUB)])
        pltpu.sync_copy(o2_v, s2_hbm.at[pl.ds(p0, SUB)])
        pltpu.sync_copy(o3_v, mg_hbm.at[pl.ds(p0, SUB)])
        pltpu.sync_copy(o4_v, mn_hbm.at[pl.ds(p0, SUB)])
        pltpu.sync_copy(o5_v, mx_hbm.at[pl.ds(p0, SUB)])
        return carry

    lax.fori_loop(0, NSUB, sub_step, 0)


@functools.cache
def _sc_call_build():
    return functools.partial(
        pl.kernel,
        mesh=plsc.VectorSubcoreMesh(core_axis_name="c", subcore_axis_name="s"),
        out_type=[jax.ShapeDtypeStruct((B * N, DIMS), jnp.float32)] * 5,
        scratch_types=[
            pltpu.VMEM((SUB * (K + 1),), jnp.int32),
            pltpu.VMEM((SUB * (K + 1), DIMS), jnp.float32),
            pltpu.VMEM((SUB * (K + 1), DIMS), jnp.float32),
            pltpu.VMEM((SUB, DIMS), jnp.float32),
            pltpu.VMEM((SUB, DIMS), jnp.float32),
            pltpu.VMEM((SUB, DIMS), jnp.float32),
            pltpu.VMEM((SUB, DIMS), jnp.float32),
            pltpu.VMEM((SUB, DIMS), jnp.float32),
            pltpu.SemaphoreType.DMA,
            pltpu.SemaphoreType.DMA,
        ],
    )(_sc_body)


def _sc_call(gt2, xt2, idxflat):
    return _sc_call_build()(gt2, xt2, idxflat)


def _kc_body(s1_ref, s2_ref, mg_ref, mn_ref, mx_ref, cc_ref, xt_ref,
             w2_ref, g1_ref, b1_ref, g2_ref, b2_ref, out_ref, y2_scr):
    cnt = jnp.float32(B * N * K)
    S1 = s1_ref[...]
    Cc = cc_ref[...]
    sum1 = jnp.sum(S1 + K * Cc, axis=0, keepdims=True)
    ey2 = jnp.sum(s2_ref[...] + 2.0 * Cc * S1 + K * Cc * Cc,
                  axis=0, keepdims=True)
    mean1 = sum1 / cnt
    var1 = ey2 / cnt - mean1 * mean1
    s1v = g1_ref[...] * lax.rsqrt(var1 + EPS)
    t1v = b1_ref[...] - mean1 * s1v
    sel = jnp.where(s1v >= 0.0, mg_ref[...], mn_ref[...])
    x1m = jnp.maximum(s1v * (sel + Cc) + t1v, 0.0)

    Mx = mx_ref[...]
    Xt = xt_ref[...]
    m2 = jnp.zeros((DIMS, 1), jnp.float32)
    q2 = jnp.zeros((DIMS, 1), jnp.float32)
    for bb in range(B):
        sl = slice(bb * N, (bb + 1) * N)
        xc = jnp.concatenate([Mx[sl, 0:H], Xt[sl, 0:H],
                              Mx[sl, H:DIMS], Xt[sl, H:DIMS],
                              x1m[sl]], axis=1)
        y2b = lax.dot_general(w2_ref[...], xc, (((1,), (1,)), ((), ())),
                              preferred_element_type=jnp.float32)
        y2_scr[bb] = y2b
        m2 = m2 + jnp.sum(y2b, axis=1, keepdims=True)
        q2 = q2 + jnp.sum(y2b * y2b, axis=1, keepdims=True)
    mean2 = m2 / jnp.float32(B * N)
    var2 = q2 / jnp.float32(B * N) - mean2 * mean2
    s2v = jnp.transpose(g2_ref[...]) * lax.rsqrt(var2 + EPS)
    t2v = jnp.transpose(b2_ref[...]) - mean2 * s2v
    for bb in range(B):
        out_ref[bb] = jnp.maximum(s2v * y2_scr[bb] + t2v, 0.0)


def _run_c(s1, s2, mg, mn, mx, cct, xt, W2, g1, b1, g2, b2):
    return pl.pallas_call(
        _kc_body,
        out_shape=jax.ShapeDtypeStruct((B, DIMS, N), jnp.float32),
        scratch_shapes=[pltpu.VMEM((B, DIMS, N), jnp.float32)],
    )(s1, s2, mg, mn, mx, cct, xt, W2, g1, b1, g2, b2)


def kernel(t, x_input, W1, g1, b1, W2, g2, b2):
    A = jnp.concatenate([W1[:, 0:H], W1[:, 2 * H:3 * H]], axis=1)
    Bm = jnp.concatenate([W1[:, H:2 * H], W1[:, 3 * H:]], axis=1)
    gt, cct, xt, idx = _run_a(x_input, A, Bm)
    gt2 = gt.reshape(B * N, DIMS)
    xt2 = xt.reshape(B * N, DIMS)
    s1, s2, mg, mn, mx = _sc_call(gt2, xt2, idx.reshape(-1))
    return _run_c(s1, s2, mg, mn, mx, cct.reshape(B * N, DIMS), xt2,
                  W2, g1.reshape(1, DIMS), b1.reshape(1, DIMS),
                  g2.reshape(1, DIMS), b2.reshape(1, DIMS))


# speedup vs baseline: 3.7972x; 1.4088x over previous
import functools

import jax
import jax.numpy as jnp
from jax import lax
from jax.experimental import pallas as pl
from jax.experimental.pallas import tpu as pltpu
from jax.experimental.pallas import tpu_sc as plsc

B, DIMS, N = 4, 512, 512
H = DIMS // 2
K = 15
EPS = 1e-5

NC, NS = 2, 16
NW = NC * NS
PTS_W = (B * N) // NW
SUB = 2
NSUB = PTS_W // SUB
RPC = SUB * (K + 1)


def _ka_body(x_ref, w1_ref, gx_ref, cct_ref, idx_ref):
    bb = pl.program_id(0)
    X = x_ref[0]
    Xlo = X[:H, :]
    Xhi = X[H:, :]
    inner = lax.dot_general(Xhi, Xhi, (((0,), (0,)), ((), ())),
                            preferred_element_type=jnp.float32)
    xx = jnp.sum(Xhi * Xhi, axis=0, keepdims=True)
    P = 2.0 * inner - xx - jnp.transpose(xx)

    iota_j = lax.broadcasted_iota(jnp.int32, (N, N), 1)
    neg = jnp.float32(-jnp.inf)
    cols = []
    for _ in range(K + 1):
        rm = jnp.max(P, axis=1, keepdims=True)
        cand = jnp.where(P == rm, iota_j, jnp.int32(N))
        am = jnp.min(cand, axis=1, keepdims=True)
        P = jnp.where(iota_j == am, neg, P)
        cols.append(am)
    idx_ref[...] = jnp.concatenate(cols, axis=1) + bb * N

    dn = (((0,), (1,)), ((), ()))
    gx_ref[0, :, 0:DIMS] = (
        lax.dot_general(Xlo, w1_ref[:, 0:H], dn,
                        preferred_element_type=jnp.float32)
        + lax.dot_general(Xhi, w1_ref[:, 2 * H:3 * H], dn,
                          preferred_element_type=jnp.float32))
    gx_ref[0, :, DIMS:2 * DIMS] = jnp.transpose(X)
    cct_ref[0] = (
        lax.dot_general(Xlo, w1_ref[:, H:2 * H], dn,
                        preferred_element_type=jnp.float32)
        + lax.dot_general(Xhi, w1_ref[:, 3 * H:], dn,
                          preferred_element_type=jnp.float32))


def _run_a(x, W1):
    return pl.pallas_call(
        _ka_body,
        grid=(B,),
        in_specs=[
            pl.BlockSpec((1, DIMS, N), lambda b: (b, 0, 0)),
            pl.BlockSpec((DIMS, 2 * DIMS), lambda b: (0, 0)),
        ],
        out_specs=[
            pl.BlockSpec((1, N, 2 * DIMS), lambda b: (b, 0, 0)),
            pl.BlockSpec((1, N, DIMS), lambda b: (b, 0, 0)),
            pl.BlockSpec((N, K + 1), lambda b: (b, 0)),
        ],
        out_shape=[
            jax.ShapeDtypeStruct((B, N, 2 * DIMS), jnp.float32),
            jax.ShapeDtypeStruct((B, N, DIMS), jnp.float32),
            jax.ShapeDtypeStruct((B * N, K + 1), jnp.int32),
        ],
    )(x, W1)


def _sc_body(gx_hbm, idx_hbm, out_hbm, idxa_v, rows_v, out_v, sg0, sg1, so0, so1):
    wid = lax.axis_index("s") * NC + lax.axis_index("c")
    base = wid * PTS_W
    pltpu.sync_copy(idx_hbm.at[pl.ds(base * (K + 1), PTS_W * (K + 1))], idxa_v)
    sgs = (sg0, sg1)
    sos = (so0, so1)

    def start_g(s, par):
        pltpu.async_copy(gx_hbm.at[idxa_v.at[pl.ds(s * RPC, RPC)]],
                         rows_v.at[par], sgs[par])

    def wait_g(par):
        pltpu.make_async_copy(gx_hbm.at[idxa_v.at[pl.ds(0, RPC)]],
                              rows_v.at[par], sgs[par]).wait()

    def start_o(s, par):
        pltpu.async_copy(out_v.at[par],
                         out_hbm.at[pl.ds(base + s * SUB, SUB)], sos[par])

    def wait_o(par):
        pltpu.make_async_copy(out_v.at[par],
                              out_hbm.at[pl.ds(0, SUB)], sos[par]).wait()

    def compute(s, par):
        def cb_step(cb, c):
            og = pl.ds(cb * 16, 16)
            for p in range(SUB):
                r0 = p * (K + 1)
                v = rows_v[par, r0 + 1, og]
                s1 = v
                s2 = v * v
                vmax = v
                vmin = v
                for kk in range(2, K + 1):
                    v = rows_v[par, r0 + kk, og]
                    s1 = s1 + v
                    s2 = s2 + v * v
                    vmax = jnp.maximum(vmax, v)
                    vmin = jnp.minimum(vmin, v)
                ox = pl.ds(DIMS + cb * 16, 16)
                u = rows_v[par, r0 + 1, ox]
                umax = u
                for kk in range(2, K + 1):
                    umax = jnp.maximum(umax, rows_v[par, r0 + kk, ox])
                out_v[par, p, pl.ds(0 * DIMS + cb * 16, 16)] = s1
                out_v[par, p, pl.ds(1 * DIMS + cb * 16, 16)] = s2
                out_v[par, p, pl.ds(2 * DIMS + cb * 16, 16)] = vmax
                out_v[par, p, pl.ds(3 * DIMS + cb * 16, 16)] = vmin
                out_v[par, p, pl.ds(4 * DIMS + cb * 16, 16)] = umax
            return c

        lax.fori_loop(0, DIMS // 16, cb_step, 0)

    start_g(0, 0)

    def pair(i, carry):
        s0 = i * 2
        wait_g(0)
        start_g(s0 + 1, 1)

        @pl.when(i > 0)
        def _():
            wait_o(0)

        compute(s0, 0)
        start_o(s0, 0)

        wait_g(1)

        @pl.when(i < NSUB // 2 - 1)
        def _():
            start_g(s0 + 2, 0)

        @pl.when(i > 0)
        def _():
            wait_o(1)

        compute(s0 + 1, 1)
        start_o(s0 + 1, 1)
        return carry

    lax.fori_loop(0, NSUB // 2, pair, 0)
    wait_o(0)
    wait_o(1)


@functools.cache
def _sc_call_build():
    return functools.partial(
        pl.kernel,
        mesh=plsc.VectorSubcoreMesh(core_axis_name="c", subcore_axis_name="s"),
        out_type=jax.ShapeDtypeStruct((B * N, 5 * DIMS), jnp.float32),
        scratch_types=[
            pltpu.VMEM((PTS_W * (K + 1),), jnp.int32),
            pltpu.VMEM((2, RPC, 2 * DIMS), jnp.float32),
            pltpu.VMEM((2, SUB, 5 * DIMS), jnp.float32),
            pltpu.SemaphoreType.DMA,
            pltpu.SemaphoreType.DMA,
            pltpu.SemaphoreType.DMA,
            pltpu.SemaphoreType.DMA,
        ],
    )(_sc_body)


def _sc_call(gx2, idxflat):
    return _sc_call_build()(gx2, idxflat)


def _kc_body(sc_ref, gx_ref, cc_ref, w2_ref, g1_ref, b1_ref, g2_ref, b2_ref,
             out_ref, y2_scr):
    cnt = jnp.float32(B * N * K)
    S1 = sc_ref[:, 0:DIMS]
    Cc = cc_ref[...]
    sum1 = jnp.sum(S1 + K * Cc, axis=0, keepdims=True)
    ey2 = jnp.sum(sc_ref[:, DIMS:2 * DIMS] + 2.0 * Cc * S1 + K * Cc * Cc,
                  axis=0, keepdims=True)
    mean1 = sum1 / cnt
    var1 = ey2 / cnt - mean1 * mean1
    s1v = g1_ref[...] * lax.rsqrt(var1 + EPS)
    t1v = b1_ref[...] - mean1 * s1v
    sel = jnp.where(s1v >= 0.0, sc_ref[:, 2 * DIMS:3 * DIMS],
                    sc_ref[:, 3 * DIMS:4 * DIMS])
    x1m = jnp.maximum(s1v * (sel + Cc) + t1v, 0.0)

    m2 = jnp.zeros((DIMS, 1), jnp.float32)
    q2 = jnp.zeros((DIMS, 1), jnp.float32)
    for bb in range(B):
        sl = slice(bb * N, (bb + 1) * N)
        Mx = sc_ref[sl, 4 * DIMS:5 * DIMS]
        Xt = gx_ref[sl, DIMS:2 * DIMS]
        xc = jnp.concatenate([Mx[:, 0:H], Xt[:, 0:H],
                              Mx[:, H:DIMS], Xt[:, H:DIMS],
                              x1m[sl]], axis=1)
        y2b = lax.dot_general(w2_ref[...], xc, (((1,), (1,)), ((), ())),
                              preferred_element_type=jnp.float32)
        y2_scr[bb] = y2b
        m2 = m2 + jnp.sum(y2b, axis=1, keepdims=True)
        q2 = q2 + jnp.sum(y2b * y2b, axis=1, keepdims=True)
    mean2 = m2 / jnp.float32(B * N)
    var2 = q2 / jnp.float32(B * N) - mean2 * mean2
    s2v = jnp.transpose(g2_ref[...]) * lax.rsqrt(var2 + EPS)
    t2v = jnp.transpose(b2_ref[...]) - mean2 * s2v
    for bb in range(B):
        out_ref[bb] = jnp.maximum(s2v * y2_scr[bb] + t2v, 0.0)


def _run_c(sc_out, gx2, cct, W2, g1, b1, g2, b2):
    return pl.pallas_call(
        _kc_body,
        out_shape=jax.ShapeDtypeStruct((B, DIMS, N), jnp.float32),
        scratch_shapes=[pltpu.VMEM((B, DIMS, N), jnp.float32)],
    )(sc_out, gx2, cct, W2, g1, b1, g2, b2)


def kernel(t, x_input, W1, g1, b1, W2, g2, b2):
    gx, cct, idx = _run_a(x_input, W1)
    gx2 = gx.reshape(B * N, 2 * DIMS)
    sc_out = _sc_call(gx2, idx.reshape(-1))
    return _run_c(sc_out, gx2, cct.reshape(B * N, DIMS),
                  W2, g1.reshape(1, DIMS), b1.reshape(1, DIMS),
                  g2.reshape(1, DIMS), b2.reshape(1, DIMS))
